# baseline (speedup 1.0000x reference)
import functools

import numpy as np
import jax
import jax.numpy as jnp
from jax import lax
from jax.experimental import pallas as pl
from jax.experimental.pallas import tpu as pltpu
from jax.experimental.pallas import tpu_sc as plsc

N_NODES_C = 50000
N_EL_C = 200000

_GP = 1.0 / np.sqrt(3.0)
_S = np.array(
    [[-1, -1, -1], [1, -1, -1], [1, 1, -1], [-1, 1, -1],
     [-1, -1, 1], [1, -1, 1], [1, 1, 1], [-1, 1, 1]], np.float64)
_XI = _S * _GP


def _build_lin_map():
    A = np.zeros((80, 64), np.float64)
    for q in range(8):
        xi = _XI[q]
        N = 0.125 * (1 + _S[:, 0] * xi[0]) * (1 + _S[:, 1] * xi[1]) * (1 + _S[:, 2] * xi[2])
        g0 = 0.125 * _S[:, 0] * (1 + _S[:, 1] * xi[1]) * (1 + _S[:, 2] * xi[2])
        g1 = 0.125 * (1 + _S[:, 0] * xi[0]) * _S[:, 1] * (1 + _S[:, 2] * xi[2])
        g2 = 0.125 * (1 + _S[:, 0] * xi[0]) * (1 + _S[:, 1] * xi[1]) * _S[:, 2]
        dN = np.stack([g0, g1, g2], axis=1)
        for k in range(9):
            i, j = divmod(k, 3)
            for a in range(8):
                A[k * 8 + q, 8 * a + j] = dN[a, i]
        for a in range(8):
            A[72 + q, 8 * a + 3] = N[a]
    return A.astype(np.float32)


_A2 = _build_lin_map()

_A128 = np.zeros((160, 128), np.float32)
_A128[0:80, 0:64] = _A2
_A128[80:160, 64:128] = _A2

_NC, _NS = 2, 16
_NW = _NC * _NS
_B = N_EL_C * 8
_PER_W = _B // _NW
_CH = 2000
_NCH = _PER_W // _CH
_NBUF = 4
_EB = 2000
_NBLK = N_EL_C // _EB


_HALF8 = (N_EL_C // 2) * 8
_ROWS_W = (N_EL_C // 2) // _NW
_RCH = _ROWS_W // _NCH
_ACH = _RCH * 8


@functools.lru_cache(maxsize=1)
def _make_sc_gather():
    mesh = plsc.VectorSubcoreMesh(
        core_axis_name="c", subcore_axis_name="s",
        num_cores=_NC, num_subcores=_NS)

    @functools.partial(
        pl.kernel,
        mesh=mesh,
        out_type=jax.ShapeDtypeStruct((_B, 8), jnp.float32),
        scratch_types=[
            *[pltpu.VMEM((_CH,), jnp.int32) for _ in range(_NBUF)],
            *[pltpu.VMEM((_CH, 8), jnp.float32) for _ in range(_NBUF)],
            pltpu.VMEM((_ACH,), jnp.int32),
            pltpu.VMEM((_ACH,), jnp.int32),
            *[pltpu.SemaphoreType.DMA for _ in range(2 * _NBUF)],
        ],
        compiler_params=pltpu.CompilerParams(
            use_tc_tiling_on_sc=False, needs_layout_passes=False),
    )
    def _sc_gather(tbl, idx_lo, idx_hi, out, *bufs):
        idx_v = bufs[:_NBUF]
        rows_v = bufs[_NBUF:2 * _NBUF]
        idx_a, idx_b = bufs[2 * _NBUF], bufs[2 * _NBUF + 1]
        gsem = bufs[2 * _NBUF + 2:3 * _NBUF + 2]
        wsem = bufs[3 * _NBUF + 2:4 * _NBUF + 2]
        wid = lax.axis_index("s") * _NC + lax.axis_index("c")
        base = wid * _PER_W
        abase = wid * _ROWS_W * 8
        lane = lax.iota(jnp.int32, 16)
        sub = lane & 7
        is_lo = lane < 8

        def stage(c, b):
            off = base + c * _CH

            @pl.when(c >= _NBUF)
            def _():
                pltpu.make_async_copy(
                    rows_v[b], out.at[pl.ds(off, _CH)], wsem[b]).wait()

            aoff = abase + c * _ACH
            pltpu.sync_copy(idx_lo.at[pl.ds(aoff, _ACH)], idx_a)
            pltpu.sync_copy(idx_hi.at[pl.ds(aoff, _ACH)], idx_b)

            def ilv(t, carry):
                ii = t * 8 + sub
                va = plsc.load_gather(idx_a, [ii])
                vb = plsc.load_gather(idx_b, [ii])
                idx_v[b][pl.ds(t * 16, 16)] = jnp.where(is_lo, va, vb)
                return carry

            lax.fori_loop(0, _RCH, ilv, 0)
            pltpu.async_copy(tbl.at[idx_v[b]], rows_v[b], gsem[b])

            b2 = (b - 2) % _NBUF

            @pl.when(c >= 2)
            def _():
                off2 = off - 2 * _CH
                pltpu.make_async_copy(
                    tbl.at[idx_v[b2]], rows_v[b2], gsem[b2]).wait()
                pltpu.async_copy(
                    rows_v[b2], out.at[pl.ds(off2, _CH)], wsem[b2])

        stage(0, 0)

        def body(g, carry):
            c0 = 1 + g * _NBUF
            for b in range(_NBUF):
                stage(c0 + b, (1 + b) % _NBUF)
            return carry

        lax.fori_loop(0, (_NCH - 1) // _NBUF, body, 0)

        for c in (_NCH - 2, _NCH - 1):
            b = c % _NBUF
            off = base + c * _CH
            pltpu.make_async_copy(
                tbl.at[idx_v[b]], rows_v[b], gsem[b]).wait()
            pltpu.async_copy(rows_v[b], out.at[pl.ds(off, _CH)], wsem[b])
        for c in range(_NCH - _NBUF, _NCH):
            b = c % _NBUF
            off = base + c * _CH
            pltpu.make_async_copy(
                rows_v[b], out.at[pl.ds(off, _CH)], wsem[b]).wait()

    return _sc_gather


_HB = 1000


def _fem_tc(a_ref, h_ref, o_ref):
    Q = lax.dot_general(
        a_ref[...], h_ref[...], (((1,), (1,)), ((), ())),
        preferred_element_type=jnp.float32,
        precision=lax.Precision.DEFAULT)
    for p in range(2):
        P = Q[80 * p:80 * (p + 1), :]
        J = [P[k * 8:(k + 1) * 8, :] for k in range(9)]
        F = P[72:80, :]
        det = (J[0] * (J[4] * J[8] - J[5] * J[7])
               - J[1] * (J[3] * J[8] - J[5] * J[6])
               + J[2] * (J[3] * J[7] - J[4] * J[6]))
        o_ref[0, p, :] = jnp.sum(F * det, axis=0)


def _tc_stage(a128, h):
    return pl.pallas_call(
        _fem_tc,
        grid=(_NBLK,),
        in_specs=[
            pl.BlockSpec((160, 128), lambda i: (0, 0)),
            pl.BlockSpec((_HB, 128), lambda i: (i, 0)),
        ],
        out_specs=pl.BlockSpec((1, 2, _HB), lambda i: (i, 0, 0)),
        out_shape=jax.ShapeDtypeStruct((_NBLK, 2, _HB), jnp.float32),
    )(a128, h)


def kernel(coords, nodal_values, elements):
    n = coords.shape[0]
    tbl = jnp.concatenate(
        [coords, nodal_values[:, None], jnp.zeros((n, 4), jnp.float32)],
        axis=1)
    el = elements.astype(jnp.int32)
    half = N_EL_C // 2
    idx_lo = el[:half].reshape(-1)
    idx_hi = el[half:].reshape(-1)
    g = _make_sc_gather()(tbl, idx_lo, idx_hi)
    h = g.reshape(N_EL_C // 2, 128)
    out = _tc_stage(jnp.asarray(_A128), h)
    return out.transpose(1, 0, 2).reshape(-1)


# speedup vs baseline: 1.2098x; 1.2098x over previous
import functools

import numpy as np
import jax
import jax.numpy as jnp
from jax import lax
from jax.experimental import pallas as pl
from jax.experimental.pallas import tpu as pltpu
from jax.experimental.pallas import tpu_sc as plsc

N_NODES_C = 50000
N_EL_C = 200000

_GP = 1.0 / np.sqrt(3.0)
_S = np.array(
    [[-1, -1, -1], [1, -1, -1], [1, 1, -1], [-1, 1, -1],
     [-1, -1, 1], [1, -1, 1], [1, 1, 1], [-1, 1, 1]], np.float64)
_XI = _S * _GP


def _build_lin_map():
    A = np.zeros((80, 64), np.float64)
    for q in range(8):
        xi = _XI[q]
        N = 0.125 * (1 + _S[:, 0] * xi[0]) * (1 + _S[:, 1] * xi[1]) * (1 + _S[:, 2] * xi[2])
        g0 = 0.125 * _S[:, 0] * (1 + _S[:, 1] * xi[1]) * (1 + _S[:, 2] * xi[2])
        g1 = 0.125 * (1 + _S[:, 0] * xi[0]) * _S[:, 1] * (1 + _S[:, 2] * xi[2])
        g2 = 0.125 * (1 + _S[:, 0] * xi[0]) * (1 + _S[:, 1] * xi[1]) * _S[:, 2]
        dN = np.stack([g0, g1, g2], axis=1)
        for k in range(9):
            i, j = divmod(k, 3)
            for a in range(8):
                A[k * 8 + q, 8 * a + j] = dN[a, i]
        for a in range(8):
            A[72 + q, 8 * a + 3] = N[a]
    return A.astype(np.float32)


_A2 = _build_lin_map()

_A128 = np.zeros((160, 128), np.float32)
_A128[0:80, 0:64] = _A2
_A128[80:160, 64:128] = _A2

_NC, _NS = 2, 16
_NW = _NC * _NS
_B = N_EL_C * 8
_PER_W = _B // _NW
_CH = 2000
_NCH = _PER_W // _CH
_NBUF = 4
_EB = 2000
_NBLK = N_EL_C // _EB


_HALF8 = (N_EL_C // 2) * 8
_ROWS_W = (N_EL_C // 2) // _NW
_RCH = _ROWS_W // _NCH
_ACH = _RCH * 8


@functools.lru_cache(maxsize=1)
def _make_sc_gather():
    mesh = plsc.VectorSubcoreMesh(
        core_axis_name="c", subcore_axis_name="s",
        num_cores=_NC, num_subcores=_NS)

    @functools.partial(
        pl.kernel,
        mesh=mesh,
        out_type=jax.ShapeDtypeStruct((_B, 8), jnp.float32),
        scratch_types=[
            *[pltpu.VMEM((_CH,), jnp.int32) for _ in range(_NBUF)],
            *[pltpu.VMEM((_CH, 8), jnp.float32) for _ in range(_NBUF)],
            pltpu.VMEM((_ACH,), jnp.int32),
            pltpu.VMEM((_ACH,), jnp.int32),
            *[pltpu.SemaphoreType.DMA for _ in range(2 * _NBUF)],
        ],
        compiler_params=pltpu.CompilerParams(
            use_tc_tiling_on_sc=False, needs_layout_passes=False),
    )
    def _sc_gather(tbl, idx, out, *bufs):
        idx_v = bufs[:_NBUF]
        rows_v = bufs[_NBUF:2 * _NBUF]
        idx_a, idx_b = bufs[2 * _NBUF], bufs[2 * _NBUF + 1]
        gsem = bufs[2 * _NBUF + 2:3 * _NBUF + 2]
        wsem = bufs[3 * _NBUF + 2:4 * _NBUF + 2]
        wid = lax.axis_index("s") * _NC + lax.axis_index("c")
        base = wid * _PER_W
        abase = wid * _ROWS_W * 8
        lane = lax.iota(jnp.int32, 16)
        sub = lane & 7
        is_lo = lane < 8

        def stage(c, b):
            off = base + c * _CH

            @pl.when(c >= _NBUF)
            def _():
                pltpu.make_async_copy(
                    rows_v[b], out.at[pl.ds(off, _CH)], wsem[b]).wait()

            aoff = abase + c * _ACH
            pltpu.sync_copy(idx.at[pl.ds(aoff, _ACH)], idx_a)
            pltpu.sync_copy(idx.at[pl.ds(_HALF8 + aoff, _ACH)], idx_b)

            def ilv(t, carry):
                ii = t * 8 + sub
                va = plsc.load_gather(idx_a, [ii])
                vb = plsc.load_gather(idx_b, [ii])
                idx_v[b][pl.ds(t * 16, 16)] = jnp.where(is_lo, va, vb)
                return carry

            lax.fori_loop(0, _RCH, ilv, 0)
            pltpu.async_copy(tbl.at[idx_v[b]], rows_v[b], gsem[b])

            b2 = (b - 2) % _NBUF

            @pl.when(c >= 2)
            def _():
                off2 = off - 2 * _CH
                pltpu.make_async_copy(
                    tbl.at[idx_v[b2]], rows_v[b2], gsem[b2]).wait()
                pltpu.async_copy(
                    rows_v[b2], out.at[pl.ds(off2, _CH)], wsem[b2])

        stage(0, 0)

        def body(g, carry):
            c0 = 1 + g * _NBUF
            for b in range(_NBUF):
                stage(c0 + b, (1 + b) % _NBUF)
            return carry

        lax.fori_loop(0, (_NCH - 1) // _NBUF, body, 0)

        for c in (_NCH - 2, _NCH - 1):
            b = c % _NBUF
            off = base + c * _CH
            pltpu.make_async_copy(
                tbl.at[idx_v[b]], rows_v[b], gsem[b]).wait()
            pltpu.async_copy(rows_v[b], out.at[pl.ds(off, _CH)], wsem[b])
        for c in range(_NCH - _NBUF, _NCH):
            b = c % _NBUF
            off = base + c * _CH
            pltpu.make_async_copy(
                rows_v[b], out.at[pl.ds(off, _CH)], wsem[b]).wait()

    return _sc_gather


_HB = 1000


def _fem_tc(a_ref, h_ref, o_ref):
    Q = lax.dot_general(
        a_ref[...], h_ref[...], (((1,), (1,)), ((), ())),
        preferred_element_type=jnp.float32,
        precision=lax.Precision.DEFAULT)
    for p in range(2):
        P = Q[80 * p:80 * (p + 1), :]
        J = [P[k * 8:(k + 1) * 8, :] for k in range(9)]
        F = P[72:80, :]
        det = (J[0] * (J[4] * J[8] - J[5] * J[7])
               - J[1] * (J[3] * J[8] - J[5] * J[6])
               + J[2] * (J[3] * J[7] - J[4] * J[6]))
        o_ref[0, p, :] = jnp.sum(F * det, axis=0)


def _tc_stage(a128, h):
    return pl.pallas_call(
        _fem_tc,
        grid=(_NBLK,),
        in_specs=[
            pl.BlockSpec((160, 128), lambda i: (0, 0)),
            pl.BlockSpec((_HB, 128), lambda i: (i, 0)),
        ],
        out_specs=pl.BlockSpec((1, 2, _HB), lambda i: (i, 0, 0)),
        out_shape=jax.ShapeDtypeStruct((_NBLK, 2, _HB), jnp.float32),
    )(a128, h)


def kernel(coords, nodal_values, elements):
    n = coords.shape[0]
    tbl = jnp.concatenate(
        [coords, nodal_values[:, None], jnp.zeros((n, 4), jnp.float32)],
        axis=1)
    idx = elements.astype(jnp.int32).reshape(-1)
    g = _make_sc_gather()(tbl, idx)
    h = g.reshape(N_EL_C // 2, 128)
    out = _tc_stage(jnp.asarray(_A128), h)
    return out.transpose(1, 0, 2).reshape(-1)
